# Initial kernel scaffold; baseline (speedup 1.0000x reference)
import jax
import jax.numpy as jnp
from jax.experimental import pallas as pl


def kernel(x, month_embed, year_embed):
    raise NotImplementedError("write your pallas kernel here")


# speedup vs baseline: 4.7552x; 4.7552x over previous
import functools

import jax
import jax.numpy as jnp
from jax import lax
from jax.experimental import pallas as pl
from jax.experimental.pallas import tpu as pltpu
from jax.experimental.pallas import tpu_sc as plsc

D_MODEL = 128
NUM_COMB = 100
_NC, _NS = 2, 16
_NW = _NC * _NS
_CHUNK = 128


def _table_body(month_ref, year_ref, out_ref):
    r = lax.broadcasted_iota(jnp.int32, (NUM_COMB, 1), 0)
    mi = r // 10
    yi = r - mi * 10
    oh_m = (mi == lax.broadcasted_iota(jnp.int32, (NUM_COMB, month_ref.shape[0]), 1)
            ).astype(jnp.float32)
    oh_y = (yi == lax.broadcasted_iota(jnp.int32, (NUM_COMB, year_ref.shape[0]), 1)
            ).astype(jnp.float32)
    out_ref[...] = (
        jnp.dot(oh_m, month_ref[...], preferred_element_type=jnp.float32)
        + jnp.dot(oh_y, year_ref[...], preferred_element_type=jnp.float32)
    )


def _build_table(month_embed, year_embed):
    return pl.pallas_call(
        _table_body,
        out_shape=jax.ShapeDtypeStruct((NUM_COMB, D_MODEL), jnp.float32),
    )(month_embed, year_embed)


def _sc_gather(table, mi, yi):
    n = mi.shape[0]
    per_w = n // _NW
    chunks = per_w // _CHUNK
    mesh = plsc.VectorSubcoreMesh(core_axis_name="c", subcore_axis_name="s")

    @functools.partial(
        pl.kernel,
        out_type=jax.ShapeDtypeStruct((n, D_MODEL), jnp.float32),
        mesh=mesh,
        scratch_types=[
            pltpu.VMEM((_CHUNK,), jnp.int32),
            pltpu.VMEM((_CHUNK,), jnp.int32),
            pltpu.VMEM((_CHUNK,), jnp.int32),
            pltpu.VMEM((_CHUNK, D_MODEL), jnp.float32),
        ],
    )
    def k(table_hbm, mi_hbm, yi_hbm, out_hbm, mi_v, yi_v, ci_v, rows_v):
        wid = lax.axis_index("s") * _NC + lax.axis_index("c")
        base_w = wid * per_w

        @pl.loop(0, chunks)
        def _chunk(g):
            base = base_w + g * _CHUNK
            pltpu.sync_copy(mi_hbm.at[pl.ds(base, _CHUNK)], mi_v)
            pltpu.sync_copy(yi_hbm.at[pl.ds(base, _CHUNK)], yi_v)

            @pl.loop(0, _CHUNK, step=16)
            def _combine(i):
                s = pl.ds(i, 16)
                ci_v[s] = mi_v[s] * 10 + yi_v[s]

            pltpu.sync_copy(table_hbm.at[ci_v], rows_v)
            pltpu.sync_copy(rows_v, out_hbm.at[pl.ds(base, _CHUNK)])

    return k(table, mi, yi)


def kernel(x, month_embed, year_embed):
    b, l, _ = x.shape
    xi = x.astype(jnp.int32)
    mi = xi[..., 0].reshape(b * l)
    yi = xi[..., 1].reshape(b * l)
    table = _build_table(month_embed, year_embed)
    out = _sc_gather(table, mi, yi)
    return out.reshape(b, l, D_MODEL)
